# baseline (speedup 1.0000x reference)
import jax
import jax.numpy as jnp
from jax.experimental import pallas as pl
from jax.experimental.pallas import tpu as pltpu

NUM_RANKS = 1000
NUM_CTX = 16
NUM_RANK_TOK = 4
MAX_TOK = 77
DIM = 768

_RB = 64
_NB = -(-NUM_RANKS // _RB)
_HEAD = 24
_TAIL = MAX_TOK - _HEAD
_TCH = 4


def _starts():
    return [min(i * _RB, NUM_RANKS - _RB) for i in range(_NB)]


def _body(ctx_ref, rank_hbm, sent_hbm, out_hbm,
          h0, h1, rk0, rk1, in_sem0, in_sem1, out_sem0, out_sem1, tail_sem):
    hs = (h0, h1)
    rks = (rk0, rk1)
    in_sems = (in_sem0, in_sem1)
    out_sems = (out_sem0, out_sem1)

    tail_cps = []
    step = -(-NUM_RANKS // _TCH)
    for c in range(_TCH):
        lo = c * step
        n = min(NUM_RANKS - lo, step)
        tail_cps.append(pltpu.make_async_copy(
            sent_hbm.at[pl.ds(lo, n), pl.ds(_HEAD, _TAIL)],
            out_hbm.at[pl.ds(lo, n), pl.ds(_HEAD, _TAIL)],
            tail_sem))
    for cp in tail_cps:
        cp.start()

    ctx_rows = jnp.broadcast_to(ctx_ref[0:15, :][None], (_RB, 15, DIM))
    h0[:, 1:16, :] = ctx_rows
    h1[:, 1:16, :] = ctx_rows

    starts = _starts()

    def in_cps(i, b):
        r0 = starts[i]
        return (
            pltpu.make_async_copy(sent_hbm.at[pl.ds(r0, _RB), pl.ds(0, 1)],
                                  hs[b].at[:, pl.ds(0, 1)], in_sems[b]),
            pltpu.make_async_copy(sent_hbm.at[pl.ds(r0, _RB), pl.ds(16, 8)],
                                  hs[b].at[:, pl.ds(16, 8)], in_sems[b]),
            pltpu.make_async_copy(rank_hbm.at[pl.ds(r0, _RB)], rks[b],
                                  in_sems[b]),
        )

    def out_cp(i, b):
        return pltpu.make_async_copy(
            hs[b], out_hbm.at[pl.ds(starts[i], _RB), pl.ds(0, _HEAD)],
            out_sems[b])

    for cp in in_cps(0, 0):
        cp.start()
    for i in range(_NB):
        b = i % 2
        for cp in in_cps(i, b):
            cp.wait()
        hs[b][:, 16:17, :] = jnp.broadcast_to(ctx_ref[15:16, :][None],
                                              (_RB, 1, DIM))
        hs[b][:, 17:21, :] = rks[b][...]
        if i >= 1:
            out_cp(i - 1, 1 - b).wait()
        if i + 1 < _NB:
            for cp in in_cps(i + 1, 1 - b):
                cp.start()
        out_cp(i, b).start()
    out_cp(_NB - 1, (_NB - 1) % 2).wait()
    for cp in tail_cps:
        cp.wait()


def kernel(context_embeds, rank_embeds, sentence_embeds):
    return pl.pallas_call(
        _body,
        in_specs=[
            pl.BlockSpec(memory_space=pltpu.VMEM),
            pl.BlockSpec(memory_space=pl.ANY),
            pl.BlockSpec(memory_space=pl.ANY),
        ],
        out_specs=pl.BlockSpec(memory_space=pl.ANY),
        out_shape=jax.ShapeDtypeStruct((NUM_RANKS, MAX_TOK, DIM), jnp.float32),
        scratch_shapes=[
            pltpu.VMEM((_RB, _HEAD, DIM), jnp.float32),
            pltpu.VMEM((_RB, _HEAD, DIM), jnp.float32),
            pltpu.VMEM((_RB, NUM_RANK_TOK, DIM), jnp.float32),
            pltpu.VMEM((_RB, NUM_RANK_TOK, DIM), jnp.float32),
            pltpu.SemaphoreType.DMA,
            pltpu.SemaphoreType.DMA,
            pltpu.SemaphoreType.DMA,
            pltpu.SemaphoreType.DMA,
            pltpu.SemaphoreType.DMA,
        ],
    )(context_embeds, rank_embeds, sentence_embeds)


# speedup vs baseline: 11.8578x; 11.8578x over previous
import jax
import jax.numpy as jnp
from jax.experimental import pallas as pl
from jax.experimental.pallas import tpu as pltpu

NUM_RANKS = 1000
NUM_CTX = 16
NUM_RANK_TOK = 4
MAX_TOK = 77
DIM = 768

_RB = 64
_NB = -(-NUM_RANKS // _RB)
_STARTS = [min(i * _RB, NUM_RANKS - _RB) for i in range(_NB)]


def _body(ctx_ref, rank_hbm, sent_hbm, out_hbm,
          h0, h1, rk0, rk1, in_sem0, in_sem1, out_sem0, out_sem1):
    hs = (h0, h1)
    rks = (rk0, rk1)
    in_sems = (in_sem0, in_sem1)
    out_sems = (out_sem0, out_sem1)

    ctx_rows = jnp.broadcast_to(ctx_ref[0:15, :][None], (_RB, 15, DIM))
    h0[:, 1:16, :] = ctx_rows
    h1[:, 1:16, :] = ctx_rows

    def in_cps(i, b):
        r0 = _STARTS[i]
        return (
            pltpu.make_async_copy(sent_hbm.at[pl.ds(r0, _RB), pl.ds(0, 1)],
                                  hs[b].at[:, pl.ds(0, 1)], in_sems[b]),
            pltpu.make_async_copy(sent_hbm.at[pl.ds(r0, _RB), pl.ds(16, 8)],
                                  hs[b].at[:, pl.ds(16, 8)], in_sems[b]),
            pltpu.make_async_copy(
                sent_hbm.at[pl.ds(r0, _RB), pl.ds(24, MAX_TOK - 24)],
                hs[b].at[:, pl.ds(24, MAX_TOK - 24)], in_sems[b]),
            pltpu.make_async_copy(rank_hbm.at[pl.ds(r0, _RB)], rks[b],
                                  in_sems[b]),
        )

    def out_cp(i, b):
        return pltpu.make_async_copy(
            hs[b], out_hbm.at[pl.ds(_STARTS[i], _RB)], out_sems[b])

    for cp in in_cps(0, 0):
        cp.start()
    for i in range(_NB):
        b = i % 2
        for cp in in_cps(i, b):
            cp.wait()
        hs[b][:, 16:17, :] = jnp.broadcast_to(ctx_ref[15:16, :][None],
                                              (_RB, 1, DIM))
        hs[b][:, 17:21, :] = rks[b][...]
        if i >= 1:
            out_cp(i - 1, 1 - b).wait()
        if i + 1 < _NB:
            for cp in in_cps(i + 1, 1 - b):
                cp.start()
        out_cp(i, b).start()
    out_cp(_NB - 1, (_NB - 1) % 2).wait()


def kernel(context_embeds, rank_embeds, sentence_embeds):
    return pl.pallas_call(
        _body,
        in_specs=[
            pl.BlockSpec(memory_space=pltpu.VMEM),
            pl.BlockSpec(memory_space=pl.ANY),
            pl.BlockSpec(memory_space=pl.ANY),
        ],
        out_specs=pl.BlockSpec(memory_space=pl.ANY),
        out_shape=jax.ShapeDtypeStruct((NUM_RANKS, MAX_TOK, DIM), jnp.float32),
        scratch_shapes=[
            pltpu.VMEM((_RB, MAX_TOK, DIM), jnp.float32),
            pltpu.VMEM((_RB, MAX_TOK, DIM), jnp.float32),
            pltpu.VMEM((_RB, NUM_RANK_TOK, DIM), jnp.float32),
            pltpu.VMEM((_RB, NUM_RANK_TOK, DIM), jnp.float32),
            pltpu.SemaphoreType.DMA,
            pltpu.SemaphoreType.DMA,
            pltpu.SemaphoreType.DMA,
            pltpu.SemaphoreType.DMA,
        ],
    )(context_embeds, rank_embeds, sentence_embeds)


# speedup vs baseline: 12.2039x; 1.0292x over previous
import jax
import jax.numpy as jnp
from jax.experimental import pallas as pl
from jax.experimental.pallas import tpu as pltpu

NUM_RANKS = 1000
NUM_CTX = 16
NUM_RANK_TOK = 4
MAX_TOK = 77
DIM = 768

_RB = 64
_NB = -(-NUM_RANKS // _RB)
_STARTS = [min(i * _RB, NUM_RANKS - _RB) for i in range(_NB)]


def _body(ctx_ref, rank_hbm, sent_hbm, out_hbm,
          h0, h1, h2, rk0, rk1, rk2,
          in_sem0, in_sem1, in_sem2, out_sem0, out_sem1, out_sem2):
    hs = (h0, h1, h2)
    rks = (rk0, rk1, rk2)
    in_sems = (in_sem0, in_sem1, in_sem2)
    out_sems = (out_sem0, out_sem1, out_sem2)

    ctx_rows = jnp.broadcast_to(ctx_ref[0:15, :][None], (_RB, 15, DIM))
    h0[:, 1:16, :] = ctx_rows
    h1[:, 1:16, :] = ctx_rows
    h2[:, 1:16, :] = ctx_rows

    def in_cps(i, b):
        r0 = _STARTS[i]
        return (
            pltpu.make_async_copy(sent_hbm.at[pl.ds(r0, _RB), pl.ds(0, 1)],
                                  hs[b].at[:, pl.ds(0, 1)], in_sems[b]),
            pltpu.make_async_copy(sent_hbm.at[pl.ds(r0, _RB), pl.ds(16, 8)],
                                  hs[b].at[:, pl.ds(16, 8)], in_sems[b]),
            pltpu.make_async_copy(
                sent_hbm.at[pl.ds(r0, _RB), pl.ds(24, MAX_TOK - 24)],
                hs[b].at[:, pl.ds(24, MAX_TOK - 24)], in_sems[b]),
            pltpu.make_async_copy(rank_hbm.at[pl.ds(r0, _RB)], rks[b],
                                  in_sems[b]),
        )

    def out_cp(i, b):
        return pltpu.make_async_copy(
            hs[b], out_hbm.at[pl.ds(_STARTS[i], _RB)], out_sems[b])

    for j in range(2):
        for cp in in_cps(j, j):
            cp.start()
    for i in range(_NB):
        b = i % 3
        for cp in in_cps(i, b):
            cp.wait()
        hs[b][:, 16:17, :] = jnp.broadcast_to(ctx_ref[15:16, :][None],
                                              (_RB, 1, DIM))
        hs[b][:, 17:21, :] = rks[b][...]
        if i >= 1:
            out_cp(i - 1, (i - 1) % 3).wait()
        if i + 2 < _NB:
            for cp in in_cps(i + 2, (i + 2) % 3):
                cp.start()
        out_cp(i, b).start()
    out_cp(_NB - 1, (_NB - 1) % 3).wait()


def kernel(context_embeds, rank_embeds, sentence_embeds):
    return pl.pallas_call(
        _body,
        in_specs=[
            pl.BlockSpec(memory_space=pltpu.VMEM),
            pl.BlockSpec(memory_space=pl.ANY),
            pl.BlockSpec(memory_space=pl.ANY),
        ],
        out_specs=pl.BlockSpec(memory_space=pl.ANY),
        out_shape=jax.ShapeDtypeStruct((NUM_RANKS, MAX_TOK, DIM), jnp.float32),
        scratch_shapes=[
            pltpu.VMEM((_RB, MAX_TOK, DIM), jnp.float32),
            pltpu.VMEM((_RB, MAX_TOK, DIM), jnp.float32),
            pltpu.VMEM((_RB, MAX_TOK, DIM), jnp.float32),
            pltpu.VMEM((_RB, NUM_RANK_TOK, DIM), jnp.float32),
            pltpu.VMEM((_RB, NUM_RANK_TOK, DIM), jnp.float32),
            pltpu.VMEM((_RB, NUM_RANK_TOK, DIM), jnp.float32),
            pltpu.SemaphoreType.DMA,
            pltpu.SemaphoreType.DMA,
            pltpu.SemaphoreType.DMA,
            pltpu.SemaphoreType.DMA,
            pltpu.SemaphoreType.DMA,
            pltpu.SemaphoreType.DMA,
        ],
    )(context_embeds, rank_embeds, sentence_embeds)
